# baseline (device time: 48748 ns/iter reference)
import jax
import jax.numpy as jnp
from jax import lax
from jax.experimental import pallas as pl
from jax.experimental.pallas import tpu as pltpu

N_DEV = 4
SUB = 4


def kernel(x, w_mat):
    m, _ = x.shape
    k_per = x.shape[1]
    _, n = w_mat.shape
    m_per = m // N_DEV
    nh = n // 2
    subw = nh // SUB

    def body(
        x_hbm, w_hbm, out_hbm,
        x_vmem, w_vmem, x_bf, w_bf, part_own,
        send_cw, recv_cw, send_ccw, recv_ccw, out_stage,
        in_sems, out_sems,
        send_sems_cw, recv_sems_cw, send_sems_ccw, recv_sems_ccw,
    ):
        my = lax.axis_index("i")
        right = lax.rem(my + 1, N_DEV)
        left = lax.rem(my + 3, N_DEV)

        blks = [left, right, lax.rem(my + 2, N_DEV), my]
        in_copies = []
        for j, c in enumerate(blks):
            cp = pltpu.make_async_copy(
                x_hbm.at[pl.ds(c * m_per, m_per), :],
                x_vmem.at[pl.ds(c * m_per, m_per), :],
                in_sems.at[j],
            )
            cp.start()
            in_copies.append(cp)
        for j, half in ((4, 0), (5, 1)):
            cp = pltpu.make_async_copy(
                w_hbm.at[:, pl.ds(half * nh, nh)],
                w_vmem.at[:, pl.ds(half * nh, nh)],
                in_sems.at[j],
            )
            cp.start()
            in_copies.append(cp)

        def wait_x(j):
            in_copies[j].wait()
            rows = pl.ds(blks[j] * m_per, m_per)
            x_bf[rows, :] = x_vmem[rows, :].astype(jnp.bfloat16)

        def wait_w(j, half):
            in_copies[j].wait()
            cols_w = pl.ds(half * nh, nh)
            w_bf[:, cols_w] = w_vmem[:, cols_w].astype(jnp.bfloat16)

        barrier_sem = pltpu.get_barrier_semaphore()
        for nbr in (left, right):
            pl.semaphore_signal(
                barrier_sem, inc=1,
                device_id=(nbr,), device_id_type=pl.DeviceIdType.MESH,
            )
        pl.semaphore_wait(barrier_sem, 2)

        def sub_dot(c, half, k):
            xb = x_bf[pl.ds(c * m_per, m_per), :]
            wh = w_bf[:, half * nh + k * subw:half * nh + (k + 1) * subw]
            return lax.dot_general(
                xb, wh, (((1,), (0,)), ((), ())),
                preferred_element_type=jnp.float32,
            )

        def make_rdma(bufs, sems, s, k, dev):
            send_buf, recv_buf = bufs
            send_sems, recv_sems = sems
            return pltpu.make_async_remote_copy(
                src_ref=send_buf.at[s, k], dst_ref=recv_buf.at[s, k],
                send_sem=send_sems.at[s, k], recv_sem=recv_sems.at[s, k],
                device_id=(dev,), device_id_type=pl.DeviceIdType.MESH,
            )

        cw_bufs = (send_cw, recv_cw)
        cw_sems = (send_sems_cw, recv_sems_cw)
        ccw_bufs = (send_ccw, recv_ccw)
        ccw_sems = (send_sems_ccw, recv_sems_ccw)

        def c_cw(s):
            return lax.rem(my + (2 * N_DEV - 1 - s), N_DEV)

        def c_ccw(s):
            return lax.rem(my + 1 + s, N_DEV)

        rdmas = []

        wait_x(0)
        wait_w(4, 0)
        for k in range(SUB):
            send_cw[0, k] = sub_dot(c_cw(0), 0, k).astype(jnp.bfloat16)
            r = make_rdma(cw_bufs, cw_sems, 0, k, right)
            r.start()
            rdmas.append(r)
        wait_x(1)
        wait_w(5, 1)
        for k in range(SUB):
            send_ccw[0, k] = sub_dot(c_ccw(0), 1, k).astype(jnp.bfloat16)
            r = make_rdma(ccw_bufs, ccw_sems, 0, k, left)
            r.start()
            rdmas.append(r)

        wait_x(2)
        wait_x(3)

        for s in (1, 2):
            for k in range(SUB):
                make_rdma(cw_bufs, cw_sems, s - 1, k, right).wait_recv()
                send_cw[s, k] = (
                    sub_dot(c_cw(s), 0, k)
                    + recv_cw[s - 1, k].astype(jnp.float32)
                ).astype(jnp.bfloat16)
                r = make_rdma(cw_bufs, cw_sems, s, k, right)
                r.start()
                rdmas.append(r)

                make_rdma(ccw_bufs, ccw_sems, s - 1, k, left).wait_recv()
                send_ccw[s, k] = (
                    sub_dot(c_ccw(s), 1, k)
                    + recv_ccw[s - 1, k].astype(jnp.float32)
                ).astype(jnp.bfloat16)
                r = make_rdma(ccw_bufs, ccw_sems, s, k, left)
                r.start()
                rdmas.append(r)

        for k in range(SUB):
            part_own[:, pl.ds(k * subw, subw)] = sub_dot(my, 0, k)
            part_own[:, pl.ds(nh + k * subw, subw)] = sub_dot(my, 1, k)

        cg = 0.7978845608028654

        def gelu(y):
            return 0.5 * y * (1.0 + jnp.tanh(cg * (y + 0.044715 * y * y * y)))

        out_copies = []
        for k in range(SUB):
            make_rdma(cw_bufs, cw_sems, 2, k, right).wait_recv()
            acc = (
                part_own[:, pl.ds(k * subw, subw)]
                + recv_cw[2, k].astype(jnp.float32)
            )
            out_stage[0, k] = gelu(acc)
            cp = pltpu.make_async_copy(
                out_stage.at[0, k],
                out_hbm.at[:, pl.ds(k * subw, subw)],
                out_sems.at[0, k],
            )
            cp.start()
            out_copies.append(cp)

            make_rdma(ccw_bufs, ccw_sems, 2, k, left).wait_recv()
            acc = (
                part_own[:, pl.ds(nh + k * subw, subw)]
                + recv_ccw[2, k].astype(jnp.float32)
            )
            out_stage[1, k] = gelu(acc)
            cp = pltpu.make_async_copy(
                out_stage.at[1, k],
                out_hbm.at[:, pl.ds(nh + k * subw, subw)],
                out_sems.at[1, k],
            )
            cp.start()
            out_copies.append(cp)

        for cp in out_copies:
            cp.wait()
        for r in rdmas:
            r.wait_send()

    return pl.pallas_call(
        body,
        out_shape=jax.ShapeDtypeStruct((m_per, n), jnp.float32),
        in_specs=[
            pl.BlockSpec(memory_space=pl.ANY),
            pl.BlockSpec(memory_space=pl.ANY),
        ],
        out_specs=pl.BlockSpec(memory_space=pl.ANY),
        scratch_shapes=[
            pltpu.VMEM((m, k_per), jnp.float32),
            pltpu.VMEM((k_per, n), jnp.float32),
            pltpu.VMEM((m, k_per), jnp.bfloat16),
            pltpu.VMEM((k_per, n), jnp.bfloat16),
            pltpu.VMEM((m_per, n), jnp.float32),
            pltpu.VMEM((3, SUB, m_per, subw), jnp.bfloat16),
            pltpu.VMEM((3, SUB, m_per, subw), jnp.bfloat16),
            pltpu.VMEM((3, SUB, m_per, subw), jnp.bfloat16),
            pltpu.VMEM((3, SUB, m_per, subw), jnp.bfloat16),
            pltpu.VMEM((2, SUB, m_per, subw), jnp.float32),
            pltpu.SemaphoreType.DMA((6,)),
            pltpu.SemaphoreType.DMA((2, SUB)),
            pltpu.SemaphoreType.DMA((3, SUB)),
            pltpu.SemaphoreType.DMA((3, SUB)),
            pltpu.SemaphoreType.DMA((3, SUB)),
            pltpu.SemaphoreType.DMA((3, SUB)),
        ],
        compiler_params=pltpu.CompilerParams(
            collective_id=0, vmem_limit_bytes=64 * 1024 * 1024,
        ),
    )(x, w_mat)


# device time: 48747 ns/iter; 1.0000x vs baseline; 1.0000x over previous
import jax
import jax.numpy as jnp
from jax import lax
from jax.experimental import pallas as pl
from jax.experimental.pallas import tpu as pltpu

N_DEV = 4
SUB = 4


def kernel(x, w_mat):
    m, _ = x.shape
    k_per = x.shape[1]
    _, n = w_mat.shape
    m_per = m // N_DEV
    nh = n // 2
    subw = nh // SUB

    def body(
        x_hbm, w_hbm, out_hbm,
        x_vmem, w_vmem, x_bf, w_bf, part_cw, part_ccw, part_own,
        send_cw, recv_cw, send_ccw, recv_ccw, out_stage,
        in_sems, out_sems,
        send_sems_cw, recv_sems_cw, send_sems_ccw, recv_sems_ccw,
    ):
        my = lax.axis_index("i")
        right = lax.rem(my + 1, N_DEV)
        left = lax.rem(my + 3, N_DEV)

        blks = [left, right, lax.rem(my + 2, N_DEV), my]
        in_copies = []
        for j, c in enumerate(blks):
            cp = pltpu.make_async_copy(
                x_hbm.at[pl.ds(c * m_per, m_per), :],
                x_vmem.at[pl.ds(c * m_per, m_per), :],
                in_sems.at[j],
            )
            cp.start()
            in_copies.append(cp)
        for j, half in ((4, 0), (5, 1)):
            cp = pltpu.make_async_copy(
                w_hbm.at[:, pl.ds(half * nh, nh)],
                w_vmem.at[:, pl.ds(half * nh, nh)],
                in_sems.at[j],
            )
            cp.start()
            in_copies.append(cp)

        def wait_x(j):
            in_copies[j].wait()
            rows = pl.ds(blks[j] * m_per, m_per)
            x_bf[rows, :] = x_vmem[rows, :].astype(jnp.bfloat16)

        def wait_w(j, half):
            in_copies[j].wait()
            cols_w = pl.ds(half * nh, nh)
            w_bf[:, cols_w] = w_vmem[:, cols_w].astype(jnp.bfloat16)

        barrier_sem = pltpu.get_barrier_semaphore()
        for nbr in (left, right):
            pl.semaphore_signal(
                barrier_sem, inc=1,
                device_id=(nbr,), device_id_type=pl.DeviceIdType.MESH,
            )
        pl.semaphore_wait(barrier_sem, 2)

        def sub_dot(c, half, k):
            xb = x_bf[pl.ds(c * m_per, m_per), :]
            wh = w_bf[:, half * nh + k * subw:half * nh + (k + 1) * subw]
            return lax.dot_general(
                xb, wh, (((1,), (0,)), ((), ())),
                preferred_element_type=jnp.float32,
            )

        def make_rdma(bufs, sems, s, k, dev):
            send_buf, recv_buf = bufs
            send_sems, recv_sems = sems
            return pltpu.make_async_remote_copy(
                src_ref=send_buf.at[s, k], dst_ref=recv_buf.at[s, k],
                send_sem=send_sems.at[s, k], recv_sem=recv_sems.at[s, k],
                device_id=(dev,), device_id_type=pl.DeviceIdType.MESH,
            )

        cw_bufs = (send_cw, recv_cw)
        cw_sems = (send_sems_cw, recv_sems_cw)
        ccw_bufs = (send_ccw, recv_ccw)
        ccw_sems = (send_sems_ccw, recv_sems_ccw)

        def c_cw(s):
            return lax.rem(my + (2 * N_DEV - 1 - s), N_DEV)

        def c_ccw(s):
            return lax.rem(my + 1 + s, N_DEV)

        rdmas = []

        wait_x(0)
        wait_w(4, 0)
        for k in range(SUB):
            send_cw[0, k] = sub_dot(c_cw(0), 0, k).astype(jnp.bfloat16)
            r = make_rdma(cw_bufs, cw_sems, 0, k, right)
            r.start()
            rdmas.append(r)
        wait_x(1)
        wait_w(5, 1)
        for k in range(SUB):
            send_ccw[0, k] = sub_dot(c_ccw(0), 1, k).astype(jnp.bfloat16)
            r = make_rdma(ccw_bufs, ccw_sems, 0, k, left)
            r.start()
            rdmas.append(r)

        wait_x(2)
        wait_x(3)

        for s in (1, 2):
            for k in range(SUB):
                part_cw[s - 1, :, pl.ds(k * subw, subw)] = sub_dot(c_cw(s), 0, k)
                part_ccw[s - 1, :, pl.ds(k * subw, subw)] = sub_dot(c_ccw(s), 1, k)
            for k in range(SUB):
                make_rdma(cw_bufs, cw_sems, s - 1, k, right).wait_recv()
                send_cw[s, k] = (
                    part_cw[s - 1, :, pl.ds(k * subw, subw)]
                    + recv_cw[s - 1, k].astype(jnp.float32)
                ).astype(jnp.bfloat16)
                r = make_rdma(cw_bufs, cw_sems, s, k, right)
                r.start()
                rdmas.append(r)

                make_rdma(ccw_bufs, ccw_sems, s - 1, k, left).wait_recv()
                send_ccw[s, k] = (
                    part_ccw[s - 1, :, pl.ds(k * subw, subw)]
                    + recv_ccw[s - 1, k].astype(jnp.float32)
                ).astype(jnp.bfloat16)
                r = make_rdma(ccw_bufs, ccw_sems, s, k, left)
                r.start()
                rdmas.append(r)

        for k in range(SUB):
            part_own[:, pl.ds(k * subw, subw)] = sub_dot(my, 0, k)
            part_own[:, pl.ds(nh + k * subw, subw)] = sub_dot(my, 1, k)

        cg = 0.7978845608028654

        def gelu(y):
            return 0.5 * y * (1.0 + jnp.tanh(cg * (y + 0.044715 * y * y * y)))

        out_copies = []
        for k in range(SUB):
            make_rdma(cw_bufs, cw_sems, 2, k, right).wait_recv()
            acc = (
                part_own[:, pl.ds(k * subw, subw)]
                + recv_cw[2, k].astype(jnp.float32)
            )
            out_stage[0, k] = gelu(acc)
            cp = pltpu.make_async_copy(
                out_stage.at[0, k],
                out_hbm.at[:, pl.ds(k * subw, subw)],
                out_sems.at[0, k],
            )
            cp.start()
            out_copies.append(cp)

            make_rdma(ccw_bufs, ccw_sems, 2, k, left).wait_recv()
            acc = (
                part_own[:, pl.ds(nh + k * subw, subw)]
                + recv_ccw[2, k].astype(jnp.float32)
            )
            out_stage[1, k] = gelu(acc)
            cp = pltpu.make_async_copy(
                out_stage.at[1, k],
                out_hbm.at[:, pl.ds(nh + k * subw, subw)],
                out_sems.at[1, k],
            )
            cp.start()
            out_copies.append(cp)

        for cp in out_copies:
            cp.wait()
        for r in rdmas:
            r.wait_send()

    return pl.pallas_call(
        body,
        out_shape=jax.ShapeDtypeStruct((m_per, n), jnp.float32),
        in_specs=[
            pl.BlockSpec(memory_space=pl.ANY),
            pl.BlockSpec(memory_space=pl.ANY),
        ],
        out_specs=pl.BlockSpec(memory_space=pl.ANY),
        scratch_shapes=[
            pltpu.VMEM((m, k_per), jnp.float32),
            pltpu.VMEM((k_per, n), jnp.float32),
            pltpu.VMEM((m, k_per), jnp.bfloat16),
            pltpu.VMEM((k_per, n), jnp.bfloat16),
            pltpu.VMEM((2, m_per, nh), jnp.float32),
            pltpu.VMEM((2, m_per, nh), jnp.float32),
            pltpu.VMEM((m_per, n), jnp.float32),
            pltpu.VMEM((3, SUB, m_per, subw), jnp.bfloat16),
            pltpu.VMEM((3, SUB, m_per, subw), jnp.bfloat16),
            pltpu.VMEM((3, SUB, m_per, subw), jnp.bfloat16),
            pltpu.VMEM((3, SUB, m_per, subw), jnp.bfloat16),
            pltpu.VMEM((2, SUB, m_per, subw), jnp.float32),
            pltpu.SemaphoreType.DMA((6,)),
            pltpu.SemaphoreType.DMA((2, SUB)),
            pltpu.SemaphoreType.DMA((3, SUB)),
            pltpu.SemaphoreType.DMA((3, SUB)),
            pltpu.SemaphoreType.DMA((3, SUB)),
            pltpu.SemaphoreType.DMA((3, SUB)),
        ],
        compiler_params=pltpu.CompilerParams(
            collective_id=0, vmem_limit_bytes=64 * 1024 * 1024,
        ),
    )(x, w_mat)


# device time: 48740 ns/iter; 1.0002x vs baseline; 1.0001x over previous
import jax
import jax.numpy as jnp
from jax import lax
from jax.experimental import pallas as pl
from jax.experimental.pallas import tpu as pltpu

N_DEV = 4
SUB = 4


def kernel(x, w_mat):
    m, _ = x.shape
    k_per = x.shape[1]
    _, n = w_mat.shape
    m_per = m // N_DEV
    nh = n // 2
    subw = nh // SUB

    def body(
        x_hbm, w_hbm, out_hbm,
        x_vmem, w_vmem, x_bf, w_bf, part_cw, part_ccw, part_own,
        send_cw, recv_cw, send_ccw, recv_ccw, out_stage,
        in_sems, out_sems,
        send_sems_cw, recv_sems_cw, send_sems_ccw, recv_sems_ccw,
    ):
        my = lax.axis_index("i")
        right = lax.rem(my + 1, N_DEV)
        left = lax.rem(my + 3, N_DEV)

        blks = [left, right, lax.rem(my + 2, N_DEV), my]
        in_copies = []
        for j, c in enumerate(blks):
            cp = pltpu.make_async_copy(
                x_hbm.at[pl.ds(c * m_per, m_per), :],
                x_vmem.at[pl.ds(c * m_per, m_per), :],
                in_sems.at[j],
            )
            cp.start()
            in_copies.append(cp)
        for j, half in ((4, 0), (5, 1)):
            cp = pltpu.make_async_copy(
                w_hbm.at[:, pl.ds(half * nh, nh)],
                w_vmem.at[:, pl.ds(half * nh, nh)],
                in_sems.at[j],
            )
            cp.start()
            in_copies.append(cp)

        def wait_x(j):
            in_copies[j].wait()
            rows = pl.ds(blks[j] * m_per, m_per)
            x_bf[rows, :] = x_vmem[rows, :].astype(jnp.bfloat16)

        def wait_w(j, half):
            in_copies[j].wait()
            cols_w = pl.ds(half * nh, nh)
            w_bf[:, cols_w] = w_vmem[:, cols_w].astype(jnp.bfloat16)

        barrier_sem = pltpu.get_barrier_semaphore()
        for nbr in (left, right):
            pl.semaphore_signal(
                barrier_sem, inc=1,
                device_id=(nbr,), device_id_type=pl.DeviceIdType.MESH,
            )
        pl.semaphore_wait(barrier_sem, 2)

        def sub_dot(c, half, k):
            xb = x_bf[pl.ds(c * m_per, m_per), :]
            wh = w_bf[:, half * nh + k * subw:half * nh + (k + 1) * subw]
            return lax.dot_general(
                xb, wh, (((1,), (0,)), ((), ())),
                preferred_element_type=jnp.float32,
            )

        def make_rdma(bufs, sems, s, k, dev):
            send_buf, recv_buf = bufs
            send_sems, recv_sems = sems
            return pltpu.make_async_remote_copy(
                src_ref=send_buf.at[s, k], dst_ref=recv_buf.at[s, k],
                send_sem=send_sems.at[s, k], recv_sem=recv_sems.at[s, k],
                device_id=(dev,), device_id_type=pl.DeviceIdType.MESH,
            )

        cw_bufs = (send_cw, recv_cw)
        cw_sems = (send_sems_cw, recv_sems_cw)
        ccw_bufs = (send_ccw, recv_ccw)
        ccw_sems = (send_sems_ccw, recv_sems_ccw)

        def c_cw(s):
            return lax.rem(my + (2 * N_DEV - 1 - s), N_DEV)

        def c_ccw(s):
            return lax.rem(my + 1 + s, N_DEV)

        rdmas = []

        wait_x(0)
        wait_w(4, 0)
        for k in range(SUB):
            send_cw[0, k] = sub_dot(c_cw(0), 0, k).astype(jnp.bfloat16)
            r = make_rdma(cw_bufs, cw_sems, 0, k, right)
            r.start()
            rdmas.append(r)
        wait_x(1)
        wait_w(5, 1)
        for k in range(SUB):
            send_ccw[0, k] = sub_dot(c_ccw(0), 1, k).astype(jnp.bfloat16)
            r = make_rdma(ccw_bufs, ccw_sems, 0, k, left)
            r.start()
            rdmas.append(r)

        wait_x(2)
        wait_x(3)

        for s in (1, 2):
            for k in range(SUB):
                part_cw[s - 1, :, pl.ds(k * subw, subw)] = sub_dot(c_cw(s), 0, k)
                part_ccw[s - 1, :, pl.ds(k * subw, subw)] = sub_dot(c_ccw(s), 1, k)
            for k in range(SUB):
                make_rdma(cw_bufs, cw_sems, s - 1, k, right).wait_recv()
                send_cw[s, k] = (
                    part_cw[s - 1, :, pl.ds(k * subw, subw)]
                    + recv_cw[s - 1, k].astype(jnp.float32)
                ).astype(jnp.bfloat16)
                r = make_rdma(cw_bufs, cw_sems, s, k, right)
                r.start()
                rdmas.append(r)

                make_rdma(ccw_bufs, ccw_sems, s - 1, k, left).wait_recv()
                send_ccw[s, k] = (
                    part_ccw[s - 1, :, pl.ds(k * subw, subw)]
                    + recv_ccw[s - 1, k].astype(jnp.float32)
                ).astype(jnp.bfloat16)
                r = make_rdma(ccw_bufs, ccw_sems, s, k, left)
                r.start()
                rdmas.append(r)

        for k in range(SUB):
            part_own[:, pl.ds(k * subw, subw)] = sub_dot(my, 0, k)
            part_own[:, pl.ds(nh + k * subw, subw)] = sub_dot(my, 1, k)

        cg = 0.7978845608028654

        def gelu(y):
            return 0.5 * y * (1.0 + jnp.tanh(cg * (y + 0.044715 * y * y * y)))

        out_copies = []
        for k in range(SUB):
            make_rdma(cw_bufs, cw_sems, 2, k, right).wait_recv()
            acc = (
                part_own[:, pl.ds(k * subw, subw)]
                + recv_cw[2, k].astype(jnp.float32)
            )
            out_stage[0, k] = gelu(acc)
            cp = pltpu.make_async_copy(
                out_stage.at[0, k],
                out_hbm.at[:, pl.ds(k * subw, subw)],
                out_sems.at[0, k],
            )
            cp.start()
            out_copies.append(cp)

            make_rdma(ccw_bufs, ccw_sems, 2, k, left).wait_recv()
            acc = (
                part_own[:, pl.ds(nh + k * subw, subw)]
                + recv_ccw[2, k].astype(jnp.float32)
            )
            out_stage[1, k] = gelu(acc)
            cp = pltpu.make_async_copy(
                out_stage.at[1, k],
                out_hbm.at[:, pl.ds(nh + k * subw, subw)],
                out_sems.at[1, k],
            )
            cp.start()
            out_copies.append(cp)

        for cp in out_copies:
            cp.wait()
        for r in rdmas:
            r.wait_send()

    return pl.pallas_call(
        body,
        out_shape=jax.ShapeDtypeStruct((m_per, n), jnp.float32),
        in_specs=[
            pl.BlockSpec(memory_space=pl.ANY),
            pl.BlockSpec(memory_space=pl.ANY),
        ],
        out_specs=pl.BlockSpec(memory_space=pltpu.MemorySpace.HBM),
        scratch_shapes=[
            pltpu.VMEM((m, k_per), jnp.float32),
            pltpu.VMEM((k_per, n), jnp.float32),
            pltpu.VMEM((m, k_per), jnp.bfloat16),
            pltpu.VMEM((k_per, n), jnp.bfloat16),
            pltpu.VMEM((2, m_per, nh), jnp.float32),
            pltpu.VMEM((2, m_per, nh), jnp.float32),
            pltpu.VMEM((m_per, n), jnp.float32),
            pltpu.VMEM((3, SUB, m_per, subw), jnp.bfloat16),
            pltpu.VMEM((3, SUB, m_per, subw), jnp.bfloat16),
            pltpu.VMEM((3, SUB, m_per, subw), jnp.bfloat16),
            pltpu.VMEM((3, SUB, m_per, subw), jnp.bfloat16),
            pltpu.VMEM((2, SUB, m_per, subw), jnp.float32),
            pltpu.SemaphoreType.DMA((6,)),
            pltpu.SemaphoreType.DMA((2, SUB)),
            pltpu.SemaphoreType.DMA((3, SUB)),
            pltpu.SemaphoreType.DMA((3, SUB)),
            pltpu.SemaphoreType.DMA((3, SUB)),
            pltpu.SemaphoreType.DMA((3, SUB)),
        ],
        compiler_params=pltpu.CompilerParams(
            collective_id=0, vmem_limit_bytes=64 * 1024 * 1024,
        ),
    )(x, w_mat)


# device time: 46962 ns/iter; 1.0380x vs baseline; 1.0379x over previous
import jax
import jax.numpy as jnp
from jax import lax
from jax.experimental import pallas as pl
from jax.experimental.pallas import tpu as pltpu

N_DEV = 4
SUB = 4


def kernel(x, w_mat):
    m, _ = x.shape
    k_per = x.shape[1]
    _, n = w_mat.shape
    m_per = m // N_DEV
    nh = n // 2
    subw = nh // SUB

    def body(
        x_hbm, w_hbm, out_hbm,
        x_vmem, w_vmem, x_bf, w_bf, part_cw, part_ccw, part_own,
        send_cw, recv_cw, send_ccw, recv_ccw, out_stage,
        in_sems, out_sems,
        send_sems_cw, recv_sems_cw, send_sems_ccw, recv_sems_ccw,
    ):
        my = lax.axis_index("i")
        right = lax.rem(my + 1, N_DEV)
        left = lax.rem(my + 3, N_DEV)

        blks = [left, right, lax.rem(my + 2, N_DEV), my]
        in_copies = {}

        def start_x_copy(j):
            cp = pltpu.make_async_copy(
                x_hbm.at[pl.ds(blks[j] * m_per, m_per), :],
                x_vmem.at[pl.ds(blks[j] * m_per, m_per), :],
                in_sems.at[j],
            )
            cp.start()
            in_copies[j] = cp

        def start_w_copy(j, half):
            cp = pltpu.make_async_copy(
                w_hbm.at[:, pl.ds(half * nh, nh)],
                w_vmem.at[:, pl.ds(half * nh, nh)],
                in_sems.at[j],
            )
            cp.start()
            in_copies[j] = cp

        start_x_copy(0)
        start_w_copy(4, 0)
        start_x_copy(1)
        start_w_copy(5, 1)
        start_x_copy(2)
        start_x_copy(3)

        def wait_x(j):
            in_copies[j].wait()
            rows = pl.ds(blks[j] * m_per, m_per)
            x_bf[rows, :] = x_vmem[rows, :].astype(jnp.bfloat16)

        def wait_w(j, half):
            in_copies[j].wait()
            cols_w = pl.ds(half * nh, nh)
            w_bf[:, cols_w] = w_vmem[:, cols_w].astype(jnp.bfloat16)

        barrier_sem = pltpu.get_barrier_semaphore()
        for nbr in (left, right):
            pl.semaphore_signal(
                barrier_sem, inc=1,
                device_id=(nbr,), device_id_type=pl.DeviceIdType.MESH,
            )
        pl.semaphore_wait(barrier_sem, 2)

        def sub_dot(c, half, k):
            xb = x_bf[pl.ds(c * m_per, m_per), :]
            wh = w_bf[:, half * nh + k * subw:half * nh + (k + 1) * subw]
            return lax.dot_general(
                xb, wh, (((1,), (0,)), ((), ())),
                preferred_element_type=jnp.float32,
            )

        def make_rdma(bufs, sems, s, k, dev):
            send_buf, recv_buf = bufs
            send_sems, recv_sems = sems
            return pltpu.make_async_remote_copy(
                src_ref=send_buf.at[s, k], dst_ref=recv_buf.at[s, k],
                send_sem=send_sems.at[s, k], recv_sem=recv_sems.at[s, k],
                device_id=(dev,), device_id_type=pl.DeviceIdType.MESH,
            )

        cw_bufs = (send_cw, recv_cw)
        cw_sems = (send_sems_cw, recv_sems_cw)
        ccw_bufs = (send_ccw, recv_ccw)
        ccw_sems = (send_sems_ccw, recv_sems_ccw)

        def c_cw(s):
            return lax.rem(my + (2 * N_DEV - 1 - s), N_DEV)

        def c_ccw(s):
            return lax.rem(my + 1 + s, N_DEV)

        rdmas = []

        wait_x(0)
        wait_w(4, 0)
        for k in range(SUB):
            send_cw[0, k] = sub_dot(c_cw(0), 0, k).astype(jnp.bfloat16)
            r = make_rdma(cw_bufs, cw_sems, 0, k, right)
            r.start()
            rdmas.append(r)
            if k == 0:
                wait_x(1)
                wait_w(5, 1)
            send_ccw[0, k] = sub_dot(c_ccw(0), 1, k).astype(jnp.bfloat16)
            r = make_rdma(ccw_bufs, ccw_sems, 0, k, left)
            r.start()
            rdmas.append(r)

        wait_x(2)
        wait_x(3)

        for s in (1, 2):
            for k in range(SUB):
                part_cw[s - 1, :, pl.ds(k * subw, subw)] = sub_dot(c_cw(s), 0, k)
                part_ccw[s - 1, :, pl.ds(k * subw, subw)] = sub_dot(c_ccw(s), 1, k)
            for k in range(SUB):
                make_rdma(cw_bufs, cw_sems, s - 1, k, right).wait_recv()
                send_cw[s, k] = (
                    part_cw[s - 1, :, pl.ds(k * subw, subw)]
                    + recv_cw[s - 1, k].astype(jnp.float32)
                ).astype(jnp.bfloat16)
                r = make_rdma(cw_bufs, cw_sems, s, k, right)
                r.start()
                rdmas.append(r)

                make_rdma(ccw_bufs, ccw_sems, s - 1, k, left).wait_recv()
                send_ccw[s, k] = (
                    part_ccw[s - 1, :, pl.ds(k * subw, subw)]
                    + recv_ccw[s - 1, k].astype(jnp.float32)
                ).astype(jnp.bfloat16)
                r = make_rdma(ccw_bufs, ccw_sems, s, k, left)
                r.start()
                rdmas.append(r)

        for k in range(SUB):
            part_own[:, pl.ds(k * subw, subw)] = sub_dot(my, 0, k)
            part_own[:, pl.ds(nh + k * subw, subw)] = sub_dot(my, 1, k)

        cg = 0.7978845608028654

        def gelu(y):
            return 0.5 * y * (1.0 + jnp.tanh(cg * (y + 0.044715 * y * y * y)))

        out_copies = []
        for k in range(SUB):
            make_rdma(cw_bufs, cw_sems, 2, k, right).wait_recv()
            acc = (
                part_own[:, pl.ds(k * subw, subw)]
                + recv_cw[2, k].astype(jnp.float32)
            )
            out_stage[0, k] = gelu(acc)
            cp = pltpu.make_async_copy(
                out_stage.at[0, k],
                out_hbm.at[:, pl.ds(k * subw, subw)],
                out_sems.at[0, k],
            )
            cp.start()
            out_copies.append(cp)

            make_rdma(ccw_bufs, ccw_sems, 2, k, left).wait_recv()
            acc = (
                part_own[:, pl.ds(nh + k * subw, subw)]
                + recv_ccw[2, k].astype(jnp.float32)
            )
            out_stage[1, k] = gelu(acc)
            cp = pltpu.make_async_copy(
                out_stage.at[1, k],
                out_hbm.at[:, pl.ds(nh + k * subw, subw)],
                out_sems.at[1, k],
            )
            cp.start()
            out_copies.append(cp)

        for cp in out_copies:
            cp.wait()
        for r in rdmas:
            r.wait_send()

    return pl.pallas_call(
        body,
        out_shape=jax.ShapeDtypeStruct((m_per, n), jnp.float32),
        in_specs=[
            pl.BlockSpec(memory_space=pl.ANY),
            pl.BlockSpec(memory_space=pl.ANY),
        ],
        out_specs=pl.BlockSpec(memory_space=pltpu.MemorySpace.HBM),
        scratch_shapes=[
            pltpu.VMEM((m, k_per), jnp.float32),
            pltpu.VMEM((k_per, n), jnp.float32),
            pltpu.VMEM((m, k_per), jnp.bfloat16),
            pltpu.VMEM((k_per, n), jnp.bfloat16),
            pltpu.VMEM((2, m_per, nh), jnp.float32),
            pltpu.VMEM((2, m_per, nh), jnp.float32),
            pltpu.VMEM((m_per, n), jnp.float32),
            pltpu.VMEM((3, SUB, m_per, subw), jnp.bfloat16),
            pltpu.VMEM((3, SUB, m_per, subw), jnp.bfloat16),
            pltpu.VMEM((3, SUB, m_per, subw), jnp.bfloat16),
            pltpu.VMEM((3, SUB, m_per, subw), jnp.bfloat16),
            pltpu.VMEM((2, SUB, m_per, subw), jnp.float32),
            pltpu.SemaphoreType.DMA((6,)),
            pltpu.SemaphoreType.DMA((2, SUB)),
            pltpu.SemaphoreType.DMA((3, SUB)),
            pltpu.SemaphoreType.DMA((3, SUB)),
            pltpu.SemaphoreType.DMA((3, SUB)),
            pltpu.SemaphoreType.DMA((3, SUB)),
        ],
        compiler_params=pltpu.CompilerParams(
            collective_id=0, vmem_limit_bytes=64 * 1024 * 1024,
        ),
    )(x, w_mat)
